# baseline (speedup 1.0000x reference)
import functools

import jax
import jax.numpy as jnp
from jax import lax
from jax.experimental import pallas as pl
from jax.experimental.pallas import tpu as pltpu
from jax.experimental.pallas import tpu_sc as plsc

N = 10000
E = 320000
D = 128

NC = 2
NS = 16
NW = NC * NS

CHUNK = 128
G8 = 8
GC = G8 * CHUNK
NP = 10240
NPT = NP // NS
EPG = 352
EP = EPG * GC
TG = EPG // NS
HG = TG // 2
WG = EPG // NW

_f32 = jnp.float32
_i32 = jnp.int32


BM = 512


def _mm1_body(x_ref, w_ref, a2_ref, h_ref, as_ref, ad_ref):
    hb = jnp.dot(x_ref[...], w_ref[...], preferred_element_type=_f32)
    h_ref[...] = hb
    av = lax.dot_general(a2_ref[...], hb, (((1,), (1,)), ((), ())))
    as_ref[...] = av[0:1]
    ad_ref[...] = av[1:2]


def _mm2_body(o0_ref, o1_ref, b_ref, w_ref, a2_ref, h_ref, as_ref, ad_ref):
    yb = o0_ref[...] + o1_ref[...] + b_ref[...]
    hb = jnp.dot(yb, w_ref[...], preferred_element_type=_f32)
    h_ref[...] = hb
    av = lax.dot_general(a2_ref[...], hb, (((1,), (1,)), ((), ())))
    as_ref[...] = av[0:1]
    ad_ref[...] = av[1:2]


def _final_body(o0_ref, o1_ref, b_ref, y_ref):
    y_ref[...] = jnp.maximum(o0_ref[...] + o1_ref[...] + b_ref[...], 0.0)


_row_spec = pl.BlockSpec((BM, D), lambda i: (i, 0))
_w_spec = pl.BlockSpec((D, D), lambda i: (0, 0))
_a2_spec = pl.BlockSpec((2, D), lambda i: (0, 0))
_b_spec = pl.BlockSpec((1, D), lambda i: (0, 0))
_av_spec = pl.BlockSpec((1, BM), lambda i: (0, i))

_mm_out = [
    jax.ShapeDtypeStruct((NP, D), _f32),
    jax.ShapeDtypeStruct((1, NP), _f32),
    jax.ShapeDtypeStruct((1, NP), _f32),
]

_mm1 = pl.pallas_call(
    _mm1_body,
    grid=(NP // BM,),
    in_specs=[_row_spec, _w_spec, _a2_spec],
    out_specs=[_row_spec, _av_spec, _av_spec],
    out_shape=_mm_out,
)

_mm2 = pl.pallas_call(
    _mm2_body,
    grid=(NP // BM,),
    in_specs=[_row_spec, _row_spec, _b_spec, _w_spec, _a2_spec],
    out_specs=[_row_spec, _av_spec, _av_spec],
    out_shape=_mm_out,
)

_final = pl.pallas_call(
    _final_body,
    grid=(NP // BM,),
    in_specs=[_row_spec, _row_spec, _b_spec],
    out_specs=_row_spec,
    out_shape=jax.ShapeDtypeStruct((NP, D), _f32),
)


_mesh = plsc.VectorSubcoreMesh(core_axis_name="c", subcore_axis_name="s")
_sc_params = pltpu.CompilerParams(use_tc_tiling_on_sc=False)


@functools.partial(
    pl.kernel,
    out_type=jax.ShapeDtypeStruct((EPG, GC), _f32),
    mesh=_mesh,
    scratch_types=[
        pltpu.VMEM((TG, GC), _i32),
        pltpu.VMEM((TG, GC), _i32),
        pltpu.VMEM((TG, GC), _f32),
        pltpu.VMEM((2, GC), _f32),
        pltpu.VMEM((2, GC), _f32),
        pltpu.VMEM((2, GC), _f32),
        pltpu.VMEM((GC,), _f32),
        pltpu.VMEM((NPT,), _f32),
        pltpu.VMEM_SHARED((NP,), _f32),
        pltpu.SemaphoreType.DMA,
        pltpu.SemaphoreType.DMA,
        pltpu.SemaphoreType.DMA,
    ],
    compiler_params=_sc_params,
)
def _sc_b1(asv, adv, srcp, dstp, attv,
           srcl, dstl, wfull, asg, adg, rd, attb, db,
           dsh, sem0, sem1, ssem):
    c = lax.axis_index("c")
    t = lax.axis_index("s")
    sem = (sem0, sem1)

    def _zero(i, _):
        db[pl.ds(i * 16, 16)] = jnp.zeros((16,), _f32)
        return 0
    lax.fori_loop(0, NPT // 16, _zero, 0)
    pltpu.sync_copy(db, dsh.at[pl.ds(t * NPT, NPT)])
    plsc.subcore_barrier()

    pltpu.sync_copy(srcp.at[pl.ds(t * TG, TG)], srcl)
    pltpu.sync_copy(dstp.at[pl.ds(t * TG, TG)], dstl)

    for b in range(2):
        pltpu.async_copy(asv.at[srcl.at[b]], asg.at[b], sem[b])
        pltpu.async_copy(adv.at[dstl.at[b]], adg.at[b], sem[b])

    def _gpair(jj, _):
        for p in range(2):
            jo = 2 * jj + p
            pltpu.make_async_copy(
                asv.at[srcl.at[jo]], asg.at[p], sem[p]).wait()
            pltpu.make_async_copy(
                adv.at[dstl.at[jo]], adg.at[p], sem[p]).wait()

            def _wblk(i, _):
                for u in range(4):
                    off = i * 64 + u * 16
                    v = (asg[p, pl.ds(off, 16)] + adg[p, pl.ds(off, 16)])
                    v = jnp.where(v >= 0.0, v, 0.2 * v)
                    wfull[jo, pl.ds(off, 16)] = jnp.exp(v)
                return 0
            lax.fori_loop(0, GC // 64, _wblk, 0)

            @pl.when(jj > 0)
            def _():
                pltpu.make_async_copy(
                    wfull.at[jo - 2], dsh.at[dstl.at[jo - 2]], ssem).wait()
            pltpu.async_copy(
                wfull.at[jo], dsh.at[dstl.at[jo]], ssem, add=True)

            @pl.when(jj + 1 < TG // 2)
            def _():
                pltpu.async_copy(
                    asv.at[srcl.at[jo + 2]], asg.at[p], sem[p])
                pltpu.async_copy(
                    adv.at[dstl.at[jo + 2]], adg.at[p], sem[p])
        return 0
    lax.fori_loop(0, TG // 2, _gpair, 0)
    for jo in (TG - 2, TG - 1):
        pltpu.make_async_copy(
            wfull.at[jo], dsh.at[dstl.at[jo]], ssem).wait()
    plsc.subcore_barrier()

    pltpu.sync_copy(dsh.at[pl.ds(t * NPT, NPT)], db)

    def _recip(i, _):
        db[pl.ds(i * 16, 16)] = 1.0 / db[pl.ds(i * 16, 16)]
        return 0
    lax.fori_loop(0, NPT // 16, _recip, 0)
    pltpu.sync_copy(db, dsh.at[pl.ds(t * NPT, NPT)])
    plsc.subcore_barrier()

    g0 = c * HG
    for b in range(2):
        pltpu.async_copy(dsh.at[dstl.at[g0 + b]], rd.at[b], sem[b])

    def _attg(jo2, p):
        gsel = jo2 + g0
        pltpu.make_async_copy(
            dsh.at[dstl.at[gsel]], rd.at[p], sem[p]).wait()

        def _ablk(i, _):
            for u in range(4):
                off = i * 64 + u * 16
                attb[pl.ds(off, 16)] = (
                    wfull[gsel, pl.ds(off, 16)] * rd[p, pl.ds(off, 16)]
                )
            return 0
        lax.fori_loop(0, GC // 64, _ablk, 0)

        @pl.when(jo2 + 2 < HG)
        def _():
            pltpu.async_copy(
                dsh.at[dstl.at[gsel + 2]], rd.at[p], sem[p])
        pltpu.sync_copy(attb, attv.at[t * TG + gsel])

    def _apair(jj2, _):
        for p in range(2):
            _attg(2 * jj2 + p, p)
        return 0
    lax.fori_loop(0, HG // 2, _apair, 0)
    _attg(HG - 1, (HG - 1) % 2)


@functools.partial(
    pl.kernel,
    out_type=[
        jax.ShapeDtypeStruct((NP, D), _f32),
        jax.ShapeDtypeStruct((NP, D), _f32),
    ],
    mesh=_mesh,
    scratch_types=[
        pltpu.VMEM((WG, GC), _i32),
        pltpu.VMEM((GC,), _i32),
        pltpu.VMEM((GC,), _f32),
        pltpu.VMEM((CHUNK, D), _f32),
        pltpu.VMEM((CHUNK, D), _f32),
        pltpu.VMEM_SHARED((NP, D), _f32),
        pltpu.SemaphoreType.DMA,
        pltpu.SemaphoreType.DMA,
    ],
    compiler_params=_sc_params,
)
def _sc_b2(h, attv, srcp, dstp, o0, o1,
           srcl, dstg, attg, hrows0, hrows1, osh, sem0, sem1):
    c = lax.axis_index("c")
    t = lax.axis_index("s")
    base = t * TG + c * HG
    hrows = (hrows0, hrows1)
    sem = (sem0, sem1)

    def _zrow(i, _):
        for k in range(D // 16):
            hrows0[i, pl.ds(k * 16, 16)] = jnp.zeros((16,), _f32)
        return 0
    lax.fori_loop(0, CHUNK, _zrow, 0)

    def _zcp(r, _):
        pltpu.sync_copy(hrows0, osh.at[pl.ds(t * NPT + r * CHUNK, CHUNK)])
        return 0
    lax.fori_loop(0, NPT // CHUNK, _zcp, 0)
    plsc.subcore_barrier()

    pltpu.sync_copy(srcp.at[pl.ds(base, WG)], srcl)

    for b in range(2):
        pltpu.async_copy(
            h.at[srcl.at[0].at[pl.ds(b * CHUNK, CHUNK)]], hrows[b], sem[b])

    def _group(jo, _):
        pltpu.sync_copy(dstp.at[base + jo], dstg)
        pltpu.sync_copy(attv.at[base + jo], attg)
        for ks in range(G8):
            b = ks % 2
            pltpu.make_async_copy(
                h.at[srcl.at[jo].at[pl.ds(ks * CHUNK, CHUNK)]],
                hrows[b], sem[b]).wait()

            def _rgroup(r, _):
                av = attg[pl.ds(ks * CHUNK + r * 16, 16)]
                rbase = r * 16
                for i16 in range(16):
                    a = jnp.zeros((16,), _f32) + av[i16]
                    for k in range(D // 16):
                        hrows[b][rbase + i16, pl.ds(k * 16, 16)] = (
                            hrows[b][rbase + i16, pl.ds(k * 16, 16)] * a
                        )
                return 0
            lax.fori_loop(0, CHUNK // 16, _rgroup, 0)

            pltpu.sync_copy(
                hrows[b], osh.at[dstg.at[pl.ds(ks * CHUNK, CHUNK)]],
                add=True)
            if ks < G8 - 2:
                pltpu.async_copy(
                    h.at[srcl.at[jo].at[pl.ds((ks + 2) * CHUNK, CHUNK)]],
                    hrows[b], sem[b])
            else:
                @pl.when(jo + 1 < WG)
                def _():
                    pltpu.async_copy(
                        h.at[srcl.at[jo + 1]
                             .at[pl.ds((ks - 6) * CHUNK, CHUNK)]],
                        hrows[b], sem[b])
        return 0
    lax.fori_loop(0, WG, _group, 0)
    plsc.subcore_barrier()

    @pl.when(c == 0)
    def _():
        def _cp(r, _):
            sl = pl.ds(t * NPT + r * CHUNK, CHUNK)
            pltpu.sync_copy(osh.at[sl], o0.at[sl])
            return 0
        lax.fori_loop(0, NPT // CHUNK, _cp, 0)

    @pl.when(c == 1)
    def _():
        def _cp(r, _):
            sl = pl.ds(t * NPT + r * CHUNK, CHUNK)
            pltpu.sync_copy(osh.at[sl], o1.at[sl])
            return 0
        lax.fori_loop(0, NPT // CHUNK, _cp, 0)


def kernel(x, e, W1, as1, ad1, b1, W2, as2, ad2, b2):
    xp = jnp.zeros((NP, D), _f32).at[:N].set(x)
    loop = jnp.arange(N, dtype=_i32)
    npad = EP - E - N
    pad = N + jnp.arange(npad, dtype=_i32) % (NP - N)
    srcp = jnp.concatenate([e[0], loop, pad]).reshape(EPG, GC)
    dstp = jnp.concatenate([e[1], loop, pad]).reshape(EPG, GC)
    a21 = jnp.stack([as1, ad1])
    a22 = jnp.stack([as2, ad2])
    b1r = b1.reshape(1, D)
    b2r = b2.reshape(1, D)

    h1, asv1, adv1 = _mm1(xp, W1, a21)
    att1 = _sc_b1(asv1.reshape(NP), adv1.reshape(NP), srcp, dstp)
    o1a, o1b = _sc_b2(h1, att1, srcp, dstp)

    h2, asv2, adv2 = _mm2(o1a, o1b, b1r, W2, a22)
    att2 = _sc_b1(asv2.reshape(NP), adv2.reshape(NP), srcp, dstp)
    o2a, o2b = _sc_b2(h2, att2, srcp, dstp)

    y = _final(o2a, o2b, b2r)
    return y[:N]


# speedup vs baseline: 1.0697x; 1.0697x over previous
import functools

import jax
import jax.numpy as jnp
from jax import lax
from jax.experimental import pallas as pl
from jax.experimental.pallas import tpu as pltpu
from jax.experimental.pallas import tpu_sc as plsc

N = 10000
E = 320000
D = 128

NC = 2
NS = 16
NW = NC * NS

CHUNK = 128
G8 = 8
GC = G8 * CHUNK
NP = 10240
NPT = NP // NS
EPG = 352
EP = EPG * GC
TG = EPG // NS
HG = TG // 2
WG = EPG // NW

_f32 = jnp.float32
_i32 = jnp.int32


BM = 512


def _mm1_body(x_ref, w_ref, a2_ref, h_ref, as_ref, ad_ref):
    hb = jnp.dot(x_ref[...], w_ref[...], preferred_element_type=_f32)
    h_ref[...] = hb
    av = lax.dot_general(a2_ref[...], hb, (((1,), (1,)), ((), ())))
    as_ref[...] = av[0:1]
    ad_ref[...] = av[1:2]


def _mm2_body(o0_ref, o1_ref, b_ref, w_ref, a2_ref, h_ref, as_ref, ad_ref):
    yb = o0_ref[...] + o1_ref[...] + b_ref[...]
    hb = jnp.dot(yb, w_ref[...], preferred_element_type=_f32)
    h_ref[...] = hb
    av = lax.dot_general(a2_ref[...], hb, (((1,), (1,)), ((), ())))
    as_ref[...] = av[0:1]
    ad_ref[...] = av[1:2]


def _final_body(o0_ref, o1_ref, b_ref, y_ref):
    y_ref[...] = jnp.maximum(o0_ref[...] + o1_ref[...] + b_ref[...], 0.0)


_row_spec = pl.BlockSpec((BM, D), lambda i: (i, 0))
_w_spec = pl.BlockSpec((D, D), lambda i: (0, 0))
_a2_spec = pl.BlockSpec((2, D), lambda i: (0, 0))
_b_spec = pl.BlockSpec((1, D), lambda i: (0, 0))
_av_spec = pl.BlockSpec((1, BM), lambda i: (0, i))

_mm_out = [
    jax.ShapeDtypeStruct((NP, D), _f32),
    jax.ShapeDtypeStruct((1, NP), _f32),
    jax.ShapeDtypeStruct((1, NP), _f32),
]

_mm1 = pl.pallas_call(
    _mm1_body,
    grid=(NP // BM,),
    in_specs=[_row_spec, _w_spec, _a2_spec],
    out_specs=[_row_spec, _av_spec, _av_spec],
    out_shape=_mm_out,
)

_mm2 = pl.pallas_call(
    _mm2_body,
    grid=(NP // BM,),
    in_specs=[_row_spec, _row_spec, _b_spec, _w_spec, _a2_spec],
    out_specs=[_row_spec, _av_spec, _av_spec],
    out_shape=_mm_out,
)

_final = pl.pallas_call(
    _final_body,
    grid=(NP // BM,),
    in_specs=[_row_spec, _row_spec, _b_spec],
    out_specs=_row_spec,
    out_shape=jax.ShapeDtypeStruct((NP, D), _f32),
)


_mesh = plsc.VectorSubcoreMesh(core_axis_name="c", subcore_axis_name="s")
_sc_params = pltpu.CompilerParams(use_tc_tiling_on_sc=False)


@functools.partial(
    pl.kernel,
    out_type=jax.ShapeDtypeStruct((EPG, GC), _f32),
    mesh=_mesh,
    scratch_types=[
        pltpu.VMEM((TG, GC), _i32),
        pltpu.VMEM((TG, GC), _i32),
        pltpu.VMEM((TG, GC), _f32),
        pltpu.VMEM((G8, CHUNK), _f32),
        pltpu.VMEM((G8, CHUNK), _f32),
        pltpu.VMEM((G8, CHUNK), _f32),
        pltpu.VMEM((GC,), _f32),
        pltpu.VMEM((NPT,), _f32),
        pltpu.VMEM_SHARED((NP,), _f32),
        pltpu.SemaphoreType.DMA,
        pltpu.SemaphoreType.DMA,
        pltpu.SemaphoreType.DMA,
        pltpu.SemaphoreType.DMA,
        pltpu.SemaphoreType.DMA,
        pltpu.SemaphoreType.DMA,
        pltpu.SemaphoreType.DMA,
        pltpu.SemaphoreType.DMA,
        pltpu.SemaphoreType.DMA,
    ],
    compiler_params=_sc_params,
)
def _sc_b1(asv, adv, srcp, dstp, attv,
           srcl, dstl, wfull, asg, adg, rd, attb, db,
           dsh, sem0, sem1, sem2, sem3, sem4, sem5, sem6, sem7, ssem):
    c = lax.axis_index("c")
    t = lax.axis_index("s")
    sem = (sem0, sem1, sem2, sem3, sem4, sem5, sem6, sem7)

    def _zero(i, _):
        db[pl.ds(i * 16, 16)] = jnp.zeros((16,), _f32)
        return 0
    lax.fori_loop(0, NPT // 16, _zero, 0)
    pltpu.sync_copy(db, dsh.at[pl.ds(t * NPT, NPT)])
    plsc.subcore_barrier()

    pltpu.sync_copy(srcp.at[pl.ds(t * TG, TG)], srcl)
    pltpu.sync_copy(dstp.at[pl.ds(t * TG, TG)], dstl)

    def _cds(ks):
        return pl.ds(ks * CHUNK, CHUNK)

    for b in range(G8):
        pltpu.async_copy(asv.at[srcl.at[0].at[_cds(b)]], asg.at[b], sem[b])
        pltpu.async_copy(adv.at[dstl.at[0].at[_cds(b)]], adg.at[b], sem[b])

    def _group(jo, _):
        @pl.when(jo > 0)
        def _():
            for ks in range(G8):
                pltpu.make_async_copy(
                    wfull.at[jo - 1].at[_cds(ks)],
                    dsh.at[dstl.at[jo - 1].at[_cds(ks)]], ssem).wait()
        for ks in range(G8):
            pltpu.make_async_copy(
                asv.at[srcl.at[jo].at[_cds(ks)]], asg.at[ks],
                sem[ks]).wait()
            pltpu.make_async_copy(
                adv.at[dstl.at[jo].at[_cds(ks)]], adg.at[ks],
                sem[ks]).wait()
            for k in range(CHUNK // 16):
                off = ks * CHUNK + k * 16
                v = (asg[ks, pl.ds(k * 16, 16)]
                     + adg[ks, pl.ds(k * 16, 16)])
                v = jnp.where(v >= 0.0, v, 0.2 * v)
                wfull[jo, pl.ds(off, 16)] = jnp.exp(v)
            pltpu.async_copy(
                wfull.at[jo].at[_cds(ks)],
                dsh.at[dstl.at[jo].at[_cds(ks)]], ssem, add=True)
            @pl.when(jo + 1 < TG)
            def _():
                pltpu.async_copy(
                    asv.at[srcl.at[jo + 1].at[_cds(ks)]], asg.at[ks],
                    sem[ks])
                pltpu.async_copy(
                    adv.at[dstl.at[jo + 1].at[_cds(ks)]], adg.at[ks],
                    sem[ks])
        return 0
    lax.fori_loop(0, TG, _group, 0)
    for ks in range(G8):
        pltpu.make_async_copy(
            wfull.at[TG - 1].at[_cds(ks)],
            dsh.at[dstl.at[TG - 1].at[_cds(ks)]], ssem).wait()
    plsc.subcore_barrier()

    pltpu.sync_copy(dsh.at[pl.ds(t * NPT, NPT)], db)

    def _recip(i, _):
        db[pl.ds(i * 16, 16)] = 1.0 / db[pl.ds(i * 16, 16)]
        return 0
    lax.fori_loop(0, NPT // 16, _recip, 0)
    pltpu.sync_copy(db, dsh.at[pl.ds(t * NPT, NPT)])
    plsc.subcore_barrier()

    g0 = c * HG
    for b in range(G8):
        pltpu.async_copy(
            dsh.at[dstl.at[g0].at[_cds(b)]], rd.at[b], sem[b])

    def _att(jo2, _):
        gsel = jo2 + g0
        for ks in range(G8):
            pltpu.make_async_copy(
                dsh.at[dstl.at[gsel].at[_cds(ks)]], rd.at[ks],
                sem[ks]).wait()
            for k in range(CHUNK // 16):
                off = ks * CHUNK + k * 16
                attb[pl.ds(off, 16)] = (
                    wfull[gsel, pl.ds(off, 16)]
                    * rd[ks, pl.ds(k * 16, 16)]
                )
            @pl.when(jo2 + 1 < HG)
            def _():
                pltpu.async_copy(
                    dsh.at[dstl.at[gsel + 1].at[_cds(ks)]], rd.at[ks],
                    sem[ks])
        pltpu.sync_copy(attb, attv.at[t * TG + gsel])
        return 0
    lax.fori_loop(0, HG, _att, 0)


@functools.partial(
    pl.kernel,
    out_type=[
        jax.ShapeDtypeStruct((NP, D), _f32),
        jax.ShapeDtypeStruct((NP, D), _f32),
    ],
    mesh=_mesh,
    scratch_types=[
        pltpu.VMEM((WG, GC), _i32),
        pltpu.VMEM((GC,), _i32),
        pltpu.VMEM((GC,), _f32),
        pltpu.VMEM((CHUNK, D), _f32),
        pltpu.VMEM((CHUNK, D), _f32),
        pltpu.VMEM_SHARED((NP, D), _f32),
        pltpu.SemaphoreType.DMA,
        pltpu.SemaphoreType.DMA,
        pltpu.SemaphoreType.DMA,
        pltpu.SemaphoreType.DMA,
    ],
    compiler_params=_sc_params,
)
def _sc_b2(h, attv, srcp, dstp, o0, o1,
           srcl, dstg, attg, hrows0, hrows1, osh, sem0, sem1, ssem0, ssem1):
    c = lax.axis_index("c")
    t = lax.axis_index("s")
    base = t * TG + c * HG
    hrows = (hrows0, hrows1)
    sem = (sem0, sem1)
    ssem = (ssem0, ssem1)

    def _zrow(i, _):
        for k in range(D // 16):
            hrows0[i, pl.ds(k * 16, 16)] = jnp.zeros((16,), _f32)
        return 0
    lax.fori_loop(0, CHUNK, _zrow, 0)

    def _zcp(r, _):
        pltpu.sync_copy(hrows0, osh.at[pl.ds(t * NPT + r * CHUNK, CHUNK)])
        return 0
    lax.fori_loop(0, NPT // CHUNK, _zcp, 0)
    plsc.subcore_barrier()

    pltpu.sync_copy(srcp.at[pl.ds(base, WG)], srcl)

    def _cds(ks):
        return pl.ds(ks * CHUNK, CHUNK)

    pltpu.async_copy(h.at[srcl.at[0].at[_cds(0)]], hrows[0], sem[0])

    def _group(jo, _):
        @pl.when(jo > 0)
        def _():
            pltpu.make_async_copy(
                hrows[1], osh.at[dstg.at[_cds(G8 - 1)]], ssem[1]).wait()
        pltpu.sync_copy(dstp.at[base + jo], dstg)
        pltpu.sync_copy(attv.at[base + jo], attg)
        for ks in range(G8):
            b = ks % 2
            ob = 1 - b
            pltpu.make_async_copy(
                h.at[srcl.at[jo].at[_cds(ks)]], hrows[b], sem[b]).wait()
            if ks < G8 - 1:
                if ks > 0:
                    pltpu.make_async_copy(
                        hrows[ob], osh.at[dstg.at[_cds(ks - 1)]],
                        ssem[ob]).wait()
                pltpu.async_copy(
                    h.at[srcl.at[jo].at[_cds(ks + 1)]], hrows[ob], sem[ob])
            else:
                @pl.when(jo + 1 < WG)
                def _():
                    pltpu.make_async_copy(
                        hrows[ob], osh.at[dstg.at[_cds(ks - 1)]],
                        ssem[ob]).wait()
                    pltpu.async_copy(
                        h.at[srcl.at[jo + 1].at[_cds(0)]], hrows[ob],
                        sem[ob])

            def _rgroup(r, _):
                av = attg[pl.ds(ks * CHUNK + r * 16, 16)]
                rbase = r * 16
                for i16 in range(16):
                    a = jnp.zeros((16,), _f32) + av[i16]
                    for k in range(D // 16):
                        hrows[b][rbase + i16, pl.ds(k * 16, 16)] = (
                            hrows[b][rbase + i16, pl.ds(k * 16, 16)] * a
                        )
                return 0
            lax.fori_loop(0, CHUNK // 16, _rgroup, 0)

            pltpu.async_copy(
                hrows[b], osh.at[dstg.at[_cds(ks)]], ssem[b], add=True)
        return 0
    lax.fori_loop(0, WG, _group, 0)
    pltpu.make_async_copy(
        hrows[0], osh.at[dstg.at[_cds(G8 - 2)]], ssem[0]).wait()
    pltpu.make_async_copy(
        hrows[1], osh.at[dstg.at[_cds(G8 - 1)]], ssem[1]).wait()
    plsc.subcore_barrier()

    @pl.when(c == 0)
    def _():
        def _cp(r, _):
            sl = pl.ds(t * NPT + r * CHUNK, CHUNK)
            pltpu.sync_copy(osh.at[sl], o0.at[sl])
            return 0
        lax.fori_loop(0, NPT // CHUNK, _cp, 0)

    @pl.when(c == 1)
    def _():
        def _cp(r, _):
            sl = pl.ds(t * NPT + r * CHUNK, CHUNK)
            pltpu.sync_copy(osh.at[sl], o1.at[sl])
            return 0
        lax.fori_loop(0, NPT // CHUNK, _cp, 0)


def kernel(x, e, W1, as1, ad1, b1, W2, as2, ad2, b2):
    xp = jnp.zeros((NP, D), _f32).at[:N].set(x)
    loop = jnp.arange(N, dtype=_i32)
    npad = EP - E - N
    pad = N + jnp.arange(npad, dtype=_i32) % (NP - N)
    srcp = jnp.concatenate([e[0], loop, pad]).reshape(EPG, GC)
    dstp = jnp.concatenate([e[1], loop, pad]).reshape(EPG, GC)
    a21 = jnp.stack([as1, ad1])
    a22 = jnp.stack([as2, ad2])
    b1r = b1.reshape(1, D)
    b2r = b2.reshape(1, D)

    h1, asv1, adv1 = _mm1(xp, W1, a21)
    att1 = _sc_b1(asv1.reshape(NP), adv1.reshape(NP), srcp, dstp)
    o1a, o1b = _sc_b2(h1, att1, srcp, dstp)

    h2, asv2, adv2 = _mm2(o1a, o1b, b1r, W2, a22)
    att2 = _sc_b1(asv2.reshape(NP), adv2.reshape(NP), srcp, dstp)
    o2a, o2b = _sc_b2(h2, att2, srcp, dstp)

    y = _final(o2a, o2b, b2r)
    return y[:N]


# speedup vs baseline: 1.3607x; 1.2720x over previous
import functools

import jax
import jax.numpy as jnp
from jax import lax
from jax.experimental import pallas as pl
from jax.experimental.pallas import tpu as pltpu
from jax.experimental.pallas import tpu_sc as plsc

N = 10000
E = 320000
D = 128

NC = 2
NS = 16
NW = NC * NS

CHUNK = 128
G8 = 8
GC = G8 * CHUNK
NP = 10240
NPT = NP // NS
EPG = 352
EP = EPG * GC
TG = EPG // NS
HG = TG // 2
WG = EPG // NW

_f32 = jnp.float32
_i32 = jnp.int32


BM = 512


def _mm1_body(x_ref, w_ref, a2_ref, h_ref, as_ref, ad_ref):
    hb = jnp.dot(x_ref[...], w_ref[...], preferred_element_type=_f32)
    h_ref[...] = hb
    av = lax.dot_general(a2_ref[...], hb, (((1,), (1,)), ((), ())))
    as_ref[...] = av[0:1]
    ad_ref[...] = av[1:2]


def _mm2_body(o0_ref, o1_ref, b_ref, w_ref, a2_ref, h_ref, as_ref, ad_ref):
    yb = o0_ref[...] + o1_ref[...] + b_ref[...]
    hb = jnp.dot(yb, w_ref[...], preferred_element_type=_f32)
    h_ref[...] = hb
    av = lax.dot_general(a2_ref[...], hb, (((1,), (1,)), ((), ())))
    as_ref[...] = av[0:1]
    ad_ref[...] = av[1:2]


def _final_body(o0_ref, o1_ref, b_ref, y_ref):
    y_ref[...] = jnp.maximum(o0_ref[...] + o1_ref[...] + b_ref[...], 0.0)


_row_spec = pl.BlockSpec((BM, D), lambda i: (i, 0))
_w_spec = pl.BlockSpec((D, D), lambda i: (0, 0))
_a2_spec = pl.BlockSpec((2, D), lambda i: (0, 0))
_b_spec = pl.BlockSpec((1, D), lambda i: (0, 0))
_av_spec = pl.BlockSpec((1, BM), lambda i: (0, i))

_mm_out = [
    jax.ShapeDtypeStruct((NP, D), _f32),
    jax.ShapeDtypeStruct((1, NP), _f32),
    jax.ShapeDtypeStruct((1, NP), _f32),
]

_mm1 = pl.pallas_call(
    _mm1_body,
    grid=(NP // BM,),
    in_specs=[_row_spec, _w_spec, _a2_spec],
    out_specs=[_row_spec, _av_spec, _av_spec],
    out_shape=_mm_out,
)

_mm2 = pl.pallas_call(
    _mm2_body,
    grid=(NP // BM,),
    in_specs=[_row_spec, _row_spec, _b_spec, _w_spec, _a2_spec],
    out_specs=[_row_spec, _av_spec, _av_spec],
    out_shape=_mm_out,
)

_final = pl.pallas_call(
    _final_body,
    grid=(NP // BM,),
    in_specs=[_row_spec, _row_spec, _b_spec],
    out_specs=_row_spec,
    out_shape=jax.ShapeDtypeStruct((NP, D), _f32),
)


_mesh = plsc.VectorSubcoreMesh(core_axis_name="c", subcore_axis_name="s")
_sc_params = pltpu.CompilerParams(use_tc_tiling_on_sc=False)


@functools.partial(
    pl.kernel,
    out_type=[
        jax.ShapeDtypeStruct((EPG, GC), _f32),
        jax.ShapeDtypeStruct((NP,), _f32),
        jax.ShapeDtypeStruct((NP,), _f32),
    ],
    mesh=_mesh,
    scratch_types=[
        pltpu.VMEM((WG, GC), _i32),
        pltpu.VMEM((WG, GC), _i32),
        pltpu.VMEM((WG, GC), _f32),
        pltpu.VMEM((G8, CHUNK), _f32),
        pltpu.VMEM((G8, CHUNK), _f32),
        pltpu.VMEM((NPT,), _f32),
        pltpu.VMEM_SHARED((NP,), _f32),
        pltpu.SemaphoreType.DMA,
        pltpu.SemaphoreType.DMA,
        pltpu.SemaphoreType.DMA,
        pltpu.SemaphoreType.DMA,
        pltpu.SemaphoreType.DMA,
        pltpu.SemaphoreType.DMA,
        pltpu.SemaphoreType.DMA,
        pltpu.SemaphoreType.DMA,
        pltpu.SemaphoreType.DMA,
        pltpu.SemaphoreType.DMA,
    ],
    compiler_params=_sc_params,
)
def _sc_b1(asv, adv, srcp, dstp, wv, d0, d1,
           srcl, dstl, wfull, asg, adg, db,
           dsh, sem0, sem1, sem2, sem3, sem4, sem5, sem6, sem7, ssem, wsem):
    c = lax.axis_index("c")
    t = lax.axis_index("s")
    sem = (sem0, sem1, sem2, sem3, sem4, sem5, sem6, sem7)
    base = t * TG + c * HG

    def _zero(i, _):
        db[pl.ds(i * 16, 16)] = jnp.zeros((16,), _f32)
        return 0
    lax.fori_loop(0, NPT // 16, _zero, 0)
    pltpu.sync_copy(db, dsh.at[pl.ds(t * NPT, NPT)])
    plsc.subcore_barrier()

    pltpu.sync_copy(srcp.at[pl.ds(base, WG)], srcl)
    pltpu.sync_copy(dstp.at[pl.ds(base, WG)], dstl)

    def _cds(ks):
        return pl.ds(ks * CHUNK, CHUNK)

    for b in range(G8):
        pltpu.async_copy(asv.at[srcl.at[0].at[_cds(b)]], asg.at[b], sem[b])
        pltpu.async_copy(adv.at[dstl.at[0].at[_cds(b)]], adg.at[b], sem[b])

    def _group(jo, _):
        @pl.when(jo > 0)
        def _():
            for ks in range(G8):
                pltpu.make_async_copy(
                    wfull.at[jo - 1].at[_cds(ks)],
                    dsh.at[dstl.at[jo - 1].at[_cds(ks)]], ssem).wait()
        for ks in range(G8):
            pltpu.make_async_copy(
                asv.at[srcl.at[jo].at[_cds(ks)]], asg.at[ks],
                sem[ks]).wait()
            pltpu.make_async_copy(
                adv.at[dstl.at[jo].at[_cds(ks)]], adg.at[ks],
                sem[ks]).wait()
            for k in range(CHUNK // 16):
                off = ks * CHUNK + k * 16
                v = (asg[ks, pl.ds(k * 16, 16)]
                     + adg[ks, pl.ds(k * 16, 16)])
                v = jnp.where(v >= 0.0, v, 0.2 * v)
                wfull[jo, pl.ds(off, 16)] = jnp.exp(v)
            pltpu.async_copy(
                wfull.at[jo].at[_cds(ks)],
                dsh.at[dstl.at[jo].at[_cds(ks)]], ssem, add=True)
            @pl.when(jo + 1 < WG)
            def _():
                pltpu.async_copy(
                    asv.at[srcl.at[jo + 1].at[_cds(ks)]], asg.at[ks],
                    sem[ks])
                pltpu.async_copy(
                    adv.at[dstl.at[jo + 1].at[_cds(ks)]], adg.at[ks],
                    sem[ks])
        pltpu.sync_copy(wfull.at[jo], wv.at[base + jo])
        return 0
    lax.fori_loop(0, WG, _group, 0)
    for ks in range(G8):
        pltpu.make_async_copy(
            wfull.at[WG - 1].at[_cds(ks)],
            dsh.at[dstl.at[WG - 1].at[_cds(ks)]], ssem).wait()
    plsc.subcore_barrier()

    sl = pl.ds(t * NPT, NPT)

    @pl.when(c == 0)
    def _():
        pltpu.sync_copy(dsh.at[sl], d0.at[sl])

    @pl.when(c == 1)
    def _():
        pltpu.sync_copy(dsh.at[sl], d1.at[sl])


@functools.partial(
    pl.kernel,
    out_type=[
        jax.ShapeDtypeStruct((NP, D), _f32),
        jax.ShapeDtypeStruct((NP, D), _f32),
    ],
    mesh=_mesh,
    scratch_types=[
        pltpu.VMEM((WG, GC), _i32),
        pltpu.VMEM((GC,), _i32),
        pltpu.VMEM((GC,), _f32),
        pltpu.VMEM((CHUNK,), _f32),
        pltpu.VMEM((CHUNK,), _f32),
        pltpu.VMEM((CHUNK, D), _f32),
        pltpu.VMEM((CHUNK, D), _f32),
        pltpu.VMEM_SHARED((NP, D), _f32),
        pltpu.SemaphoreType.DMA,
        pltpu.SemaphoreType.DMA,
        pltpu.SemaphoreType.DMA,
        pltpu.SemaphoreType.DMA,
    ],
    compiler_params=_sc_params,
)
def _sc_b2(h, attv, d0, d1, srcp, dstp, o0, o1,
           srcl, dstg, attg, den0, den1, hrows0, hrows1, osh,
           sem0, sem1, ssem0, ssem1):
    c = lax.axis_index("c")
    t = lax.axis_index("s")
    base = t * TG + c * HG
    hrows = (hrows0, hrows1)
    sem = (sem0, sem1)
    ssem = (ssem0, ssem1)

    def _zrow(i, _):
        for k in range(D // 16):
            hrows0[i, pl.ds(k * 16, 16)] = jnp.zeros((16,), _f32)
        return 0
    lax.fori_loop(0, CHUNK, _zrow, 0)

    def _zcp(r, _):
        pltpu.sync_copy(hrows0, osh.at[pl.ds(t * NPT + r * CHUNK, CHUNK)])
        return 0
    lax.fori_loop(0, NPT // CHUNK, _zcp, 0)
    plsc.subcore_barrier()

    pltpu.sync_copy(srcp.at[pl.ds(base, WG)], srcl)

    def _cds(ks):
        return pl.ds(ks * CHUNK, CHUNK)

    pltpu.async_copy(h.at[srcl.at[0].at[_cds(0)]], hrows[0], sem[0])

    def _group(jo, _):
        @pl.when(jo > 0)
        def _():
            pltpu.make_async_copy(
                hrows[1], osh.at[dstg.at[_cds(G8 - 1)]], ssem[1]).wait()
        pltpu.sync_copy(dstp.at[base + jo], dstg)
        pltpu.sync_copy(attv.at[base + jo], attg)
        for ks in range(G8):
            b = ks % 2
            ob = 1 - b
            pltpu.make_async_copy(
                h.at[srcl.at[jo].at[_cds(ks)]], hrows[b], sem[b]).wait()
            if ks < G8 - 1:
                if ks > 0:
                    pltpu.make_async_copy(
                        hrows[ob], osh.at[dstg.at[_cds(ks - 1)]],
                        ssem[ob]).wait()
                pltpu.async_copy(
                    h.at[srcl.at[jo].at[_cds(ks + 1)]], hrows[ob], sem[ob])
            else:
                @pl.when(jo + 1 < WG)
                def _():
                    pltpu.make_async_copy(
                        hrows[ob], osh.at[dstg.at[_cds(ks - 1)]],
                        ssem[ob]).wait()
                    pltpu.async_copy(
                        h.at[srcl.at[jo + 1].at[_cds(0)]], hrows[ob],
                        sem[ob])

            def _rgroup(r, _):
                av = attg[pl.ds(ks * CHUNK + r * 16, 16)]
                rbase = r * 16
                for i16 in range(16):
                    a = jnp.zeros((16,), _f32) + av[i16]
                    for k in range(D // 16):
                        hrows[b][rbase + i16, pl.ds(k * 16, 16)] = (
                            hrows[b][rbase + i16, pl.ds(k * 16, 16)] * a
                        )
                return 0
            lax.fori_loop(0, CHUNK // 16, _rgroup, 0)

            pltpu.async_copy(
                hrows[b], osh.at[dstg.at[_cds(ks)]], ssem[b], add=True)
        return 0
    lax.fori_loop(0, WG, _group, 0)
    pltpu.make_async_copy(
        hrows[0], osh.at[dstg.at[_cds(G8 - 2)]], ssem[0]).wait()
    pltpu.make_async_copy(
        hrows[1], osh.at[dstg.at[_cds(G8 - 1)]], ssem[1]).wait()
    plsc.subcore_barrier()

    def _dump(r, _):
        sl = pl.ds(t * NPT + r * CHUNK, CHUNK)
        pltpu.sync_copy(osh.at[sl], hrows0)
        pltpu.sync_copy(d0.at[sl], den0)
        pltpu.sync_copy(d1.at[sl], den1)

        def _rblk(g, _):
            rv = 1.0 / (den0[pl.ds(g * 16, 16)] + den1[pl.ds(g * 16, 16)])
            den0[pl.ds(g * 16, 16)] = rv
            return 0
        lax.fori_loop(0, CHUNK // 16, _rblk, 0)

        def _sgroup(g, _):
            av = den0[pl.ds(g * 16, 16)]
            rbase = g * 16
            for i16 in range(16):
                a = jnp.zeros((16,), _f32) + av[i16]
                for k in range(D // 16):
                    hrows0[rbase + i16, pl.ds(k * 16, 16)] = (
                        hrows0[rbase + i16, pl.ds(k * 16, 16)] * a
                    )
            return 0
        lax.fori_loop(0, CHUNK // 16, _sgroup, 0)

        @pl.when(c == 0)
        def _():
            pltpu.sync_copy(hrows0, o0.at[sl])

        @pl.when(c == 1)
        def _():
            pltpu.sync_copy(hrows0, o1.at[sl])
        return 0
    lax.fori_loop(0, NPT // CHUNK, _dump, 0)


def kernel(x, e, W1, as1, ad1, b1, W2, as2, ad2, b2):
    xp = jnp.zeros((NP, D), _f32).at[:N].set(x)
    loop = jnp.arange(N, dtype=_i32)
    npad = EP - E - N
    pad = N + jnp.arange(npad, dtype=_i32) % (NP - N)
    srcp = jnp.concatenate([e[0], loop, pad]).reshape(EPG, GC)
    dstp = jnp.concatenate([e[1], loop, pad]).reshape(EPG, GC)
    a21 = jnp.stack([as1, ad1])
    a22 = jnp.stack([as2, ad2])
    b1r = b1.reshape(1, D)
    b2r = b2.reshape(1, D)

    h1, asv1, adv1 = _mm1(xp, W1, a21)
    w1, d01, d11 = _sc_b1(asv1.reshape(NP), adv1.reshape(NP), srcp, dstp)
    o1a, o1b = _sc_b2(h1, w1, d01, d11, srcp, dstp)

    h2, asv2, adv2 = _mm2(o1a, o1b, b1r, W2, a22)
    w2, d02, d12 = _sc_b1(asv2.reshape(NP), adv2.reshape(NP), srcp, dstp)
    o2a, o2b = _sc_b2(h2, w2, d02, d12, srcp, dstp)

    y = _final(o2a, o2b, b2r)
    return y[:N]
